# baseline (speedup 1.0000x reference)
import functools
import jax
import jax.numpy as jnp
from jax import lax
from jax.experimental import pallas as pl
from jax.experimental.pallas import tpu as pltpu
from jax.experimental.pallas import tpu_sc as plsc

N_NODES = 10000
N_EDGES = 320000
D_FEAT = 128
D_HALF = D_FEAT // 2

NUM_TILES = 16
CHUNK = 40
EDGES_PER_TILE = N_EDGES // NUM_TILES
NCH = EDGES_PER_TILE // CHUNK
NB = 10
NT = NCH // NB
N_PAD = 10240
ROWS_PER_TILE = N_PAD // NUM_TILES
ZROWS = 80


def _drain(tbl_hbm, rows_b, sem):
    pltpu.make_async_copy(tbl_hbm.at[pl.ds(0, CHUNK)], rows_b, sem).wait()


def _sc_body(tbl_hbm, src_hbm, dst_hbm, out_hbm, idx_s, idx_d, rows, zbuf,
             acc, gs, ss, s):
    pltpu.sync_copy(src_hbm.at[s], idx_s)
    for k in range(NB - 1):
        pltpu.async_copy(tbl_hbm.at[idx_s.at[k]], rows[k], gs[k])
    pltpu.sync_copy(dst_hbm.at[s], idx_d)

    zeros16 = jnp.zeros((16,), jnp.float32)
    def _zstore(i, _):
        zbuf[i // (D_HALF // 16), pl.ds((i % (D_HALF // 16)) * 16, 16)] = zeros16
        return 0
    lax.fori_loop(0, ZROWS * (D_HALF // 16), _zstore, 0)
    base = s * ROWS_PER_TILE
    for k in range(ROWS_PER_TILE // ZROWS):
        pltpu.sync_copy(zbuf, acc.at[pl.ds(base + k * ZROWS, ZROWS)])
    plsc.subcore_barrier()

    def _slot(j, k, start_g, wait_s):
        _drain(tbl_hbm, rows[k], gs[k])
        pltpu.async_copy(rows[k], acc.at[idx_d.at[j]], ss[k], add=True)
        if start_g:
            bn = (k + NB - 1) % NB
            if wait_s:
                _drain(tbl_hbm, rows[bn], ss[bn])
            pltpu.async_copy(tbl_hbm.at[idx_s.at[j + NB - 1]], rows[bn],
                             gs[bn])

    for k in range(NB):
        _slot(k, k, start_g=True, wait_s=(k >= 1))

    def _turn(t, _):
        for k in range(NB):
            _slot(t * NB + k, k, start_g=True, wait_s=True)
        return 0
    lax.fori_loop(1, NT - 1, _turn, 0)

    for k in range(NB):
        _slot((NT - 1) * NB + k, k, start_g=(k == 0), wait_s=True)
    for k in range(NB):
        _drain(tbl_hbm, rows[k], ss[k])

    plsc.subcore_barrier()
    pltpu.sync_copy(acc.at[pl.ds(base, ROWS_PER_TILE)],
                    out_hbm.at[pl.ds(base, ROWS_PER_TILE)])


@functools.partial(
    pl.kernel,
    out_type=(jax.ShapeDtypeStruct((N_PAD, D_HALF), jnp.float32),
              jax.ShapeDtypeStruct((N_PAD, D_HALF), jnp.float32)),
    mesh=plsc.VectorSubcoreMesh(core_axis_name="c", subcore_axis_name="s"),
    compiler_params=pltpu.CompilerParams(use_tc_tiling_on_sc=False),
    scratch_types=(
        [pltpu.VMEM((NCH, CHUNK), jnp.int32)] * 2 +
        [pltpu.VMEM((CHUNK, D_HALF), jnp.float32)] * NB +
        [pltpu.VMEM((ZROWS, D_HALF), jnp.float32),
         pltpu.VMEM_SHARED((N_PAD, D_HALF), jnp.float32)] +
        [pltpu.SemaphoreType.DMA] * (2 * NB)
    ),
)
def _mp_kernel(xlo_hbm, xhi_hbm, src_hbm, dst_hbm, outlo_hbm, outhi_hbm,
               *refs):
    c = lax.axis_index("c")
    s = lax.axis_index("s")
    idx_s, idx_d = refs[0], refs[1]
    rows = refs[2:2 + NB]
    zbuf, acc = refs[2 + NB], refs[3 + NB]
    gs = refs[4 + NB:4 + 2 * NB]
    ss = refs[4 + 2 * NB:4 + 3 * NB]

    @pl.when(c == 0)
    def _():
        _sc_body(xlo_hbm, src_hbm, dst_hbm, outlo_hbm, idx_s, idx_d, rows,
                 zbuf, acc, gs, ss, s)

    @pl.when(c == 1)
    def _():
        _sc_body(xhi_hbm, src_hbm, dst_hbm, outhi_hbm, idx_s, idx_d, rows,
                 zbuf, acc, gs, ss, s)


def kernel(edge_index, x):
    xlo = x[:, :D_HALF]
    xhi = x[:, D_HALF:]
    src3d = edge_index[0].reshape(NUM_TILES, NCH, CHUNK)
    dst3d = edge_index[1].reshape(NUM_TILES, NCH, CHUNK)
    outlo, outhi = _mp_kernel(xlo, xhi, src3d, dst3d)
    return jnp.concatenate([outlo[:N_NODES], outhi[:N_NODES]], axis=1)


# speedup vs baseline: 1.2901x; 1.2901x over previous
import functools
import jax
import jax.numpy as jnp
from jax import lax
from jax.experimental import pallas as pl
from jax.experimental.pallas import tpu as pltpu
from jax.experimental.pallas import tpu_sc as plsc

N_NODES = 10000
N_EDGES = 320000
D_FEAT = 128
D_HALF = D_FEAT // 2

NUM_TILES = 16
CHUNK = 80
EDGES_PER_TILE = N_EDGES // NUM_TILES
NCH = EDGES_PER_TILE // CHUNK
NB = 5
NT = NCH // NB
N_PAD = 10240
ROWS_PER_TILE = N_PAD // NUM_TILES
LAST_ROWS = N_NODES - 15 * ROWS_PER_TILE
ZROWS = 80


def _drain(tbl_hbm, rows_b, sem):
    pltpu.make_async_copy(tbl_hbm.at[pl.ds(0, CHUNK)], rows_b, sem).wait()


@functools.partial(
    pl.kernel,
    out_type=jax.ShapeDtypeStruct((N_NODES, D_FEAT), jnp.float32),
    mesh=plsc.VectorSubcoreMesh(core_axis_name="c", subcore_axis_name="s"),
    compiler_params=pltpu.CompilerParams(use_tc_tiling_on_sc=False),
    scratch_types=(
        [pltpu.VMEM((NCH, CHUNK), jnp.int32)] * 2 +
        [pltpu.VMEM((CHUNK, D_HALF), jnp.float32)] * NB +
        [pltpu.VMEM((ZROWS, D_HALF), jnp.float32),
         pltpu.VMEM_SHARED((N_PAD, D_HALF), jnp.float32)] +
        [pltpu.SemaphoreType.DMA] * (2 * NB)
    ),
)
def _mp_kernel(e4d_hbm, x2_hbm, out_hbm, *refs):
    c = lax.axis_index("c")
    s = lax.axis_index("s")
    idx_s, idx_d = refs[0], refs[1]
    rows = refs[2:2 + NB]
    zbuf, acc = refs[2 + NB], refs[3 + NB]
    gs = refs[4 + NB:4 + 2 * NB]
    ss = refs[4 + 2 * NB:4 + 3 * NB]

    def _xform(j):
        for q in range(CHUNK // 16):
            v = idx_s[j, pl.ds(q * 16, 16)]
            idx_s[j, pl.ds(q * 16, 16)] = v + v + c

    pltpu.sync_copy(e4d_hbm.at[0, s], idx_s)
    for k in range(NB - 1):
        _xform(k)
        pltpu.async_copy(x2_hbm.at[idx_s.at[k]], rows[k], gs[k])
    pltpu.sync_copy(e4d_hbm.at[1, s], idx_d)

    zeros16 = jnp.zeros((16,), jnp.float32)
    def _zstore(i, _):
        zbuf[i // (D_HALF // 16), pl.ds((i % (D_HALF // 16)) * 16, 16)] = zeros16
        return 0
    lax.fori_loop(0, ZROWS * (D_HALF // 16), _zstore, 0)
    base = s * ROWS_PER_TILE
    for k in range(ROWS_PER_TILE // ZROWS):
        pltpu.sync_copy(zbuf, acc.at[pl.ds(base + k * ZROWS, ZROWS)])
    plsc.subcore_barrier()

    def _slot(j, k, start_g, wait_s):
        _drain(x2_hbm, rows[k], gs[k])
        pltpu.async_copy(rows[k], acc.at[idx_d.at[j]], ss[k], add=True)
        if start_g:
            bn = (k + NB - 1) % NB
            if wait_s:
                _drain(x2_hbm, rows[bn], ss[bn])
            _xform(j + NB - 1)
            pltpu.async_copy(x2_hbm.at[idx_s.at[j + NB - 1]], rows[bn],
                             gs[bn])

    for k in range(NB):
        _slot(k, k, start_g=True, wait_s=(k >= 1))

    def _turn(t, _):
        for k in range(NB):
            _slot(t * NB + k, k, start_g=True, wait_s=True)
        return 0
    lax.fori_loop(1, NT - 1, _turn, 0)

    for k in range(NB):
        _slot((NT - 1) * NB + k, k, start_g=(k == 0), wait_s=True)
    for k in range(NB):
        _drain(x2_hbm, rows[k], ss[k])

    plsc.subcore_barrier()
    col = c * D_HALF

    @pl.when(s < NUM_TILES - 1)
    def _():
        pltpu.sync_copy(acc.at[pl.ds(base, ROWS_PER_TILE)],
                        out_hbm.at[pl.ds(base, ROWS_PER_TILE),
                                   pl.ds(col, D_HALF)])

    @pl.when(s == NUM_TILES - 1)
    def _():
        pltpu.sync_copy(acc.at[pl.ds(base, LAST_ROWS)],
                        out_hbm.at[pl.ds(base, LAST_ROWS),
                                   pl.ds(col, D_HALF)])


def kernel(edge_index, x):
    e4d = edge_index.reshape(2, NUM_TILES, NCH, CHUNK)
    x2 = x.reshape(2 * N_NODES, D_HALF)
    return _mp_kernel(e4d, x2)


# speedup vs baseline: 1.2919x; 1.0014x over previous
import functools
import jax
import jax.numpy as jnp
from jax import lax
from jax.experimental import pallas as pl
from jax.experimental.pallas import tpu as pltpu
from jax.experimental.pallas import tpu_sc as plsc

N_NODES = 10000
N_EDGES = 320000
D_FEAT = 128
D_HALF = D_FEAT // 2

NUM_TILES = 16
CHUNK = 80
EDGES_PER_TILE = N_EDGES // NUM_TILES
NCH = EDGES_PER_TILE // CHUNK
NB = 5
NT = NCH // NB
N_PAD = 10240
ROWS_PER_TILE = N_PAD // NUM_TILES
LAST_ROWS = N_NODES - 15 * ROWS_PER_TILE
ZROWS = 80


def _drain(tbl_hbm, rows_b, sem):
    pltpu.make_async_copy(tbl_hbm.at[pl.ds(0, CHUNK)], rows_b, sem).wait()


@functools.partial(
    pl.kernel,
    out_type=jax.ShapeDtypeStruct((N_NODES, D_FEAT), jnp.float32),
    mesh=plsc.VectorSubcoreMesh(core_axis_name="c", subcore_axis_name="s"),
    compiler_params=pltpu.CompilerParams(use_tc_tiling_on_sc=False),
    scratch_types=(
        [pltpu.VMEM((EDGES_PER_TILE,), jnp.int32)] * 2 +
        [pltpu.VMEM((CHUNK, D_HALF), jnp.float32)] * NB +
        [pltpu.VMEM((ZROWS, D_HALF), jnp.float32),
         pltpu.VMEM_SHARED((N_PAD, D_HALF), jnp.float32)] +
        [pltpu.SemaphoreType.DMA] * (2 * NB)
    ),
)
def _mp_kernel(e1d_hbm, x2_hbm, out_hbm, *refs):
    c = lax.axis_index("c")
    s = lax.axis_index("s")
    idx_s, idx_d = refs[0], refs[1]
    rows = refs[2:2 + NB]
    zbuf, acc = refs[2 + NB], refs[3 + NB]
    gs = refs[4 + NB:4 + 2 * NB]
    ss = refs[4 + 2 * NB:4 + 3 * NB]

    def _xform(j):
        for q in range(CHUNK // 16):
            v = idx_s[pl.ds(j * CHUNK + q * 16, 16)]
            idx_s[pl.ds(j * CHUNK + q * 16, 16)] = v + v + c

    pltpu.sync_copy(e1d_hbm.at[pl.ds(s * EDGES_PER_TILE, EDGES_PER_TILE)],
                    idx_s)
    for k in range(NB - 1):
        _xform(k)
        pltpu.async_copy(x2_hbm.at[idx_s.at[pl.ds(k * CHUNK, CHUNK)]],
                         rows[k], gs[k])
    pltpu.sync_copy(
        e1d_hbm.at[pl.ds(N_EDGES + s * EDGES_PER_TILE, EDGES_PER_TILE)], idx_d)

    zeros16 = jnp.zeros((16,), jnp.float32)
    def _zstore(i, _):
        zbuf[i // (D_HALF // 16), pl.ds((i % (D_HALF // 16)) * 16, 16)] = zeros16
        return 0
    lax.fori_loop(0, ZROWS * (D_HALF // 16), _zstore, 0)
    base = s * ROWS_PER_TILE
    for k in range(ROWS_PER_TILE // ZROWS):
        pltpu.sync_copy(zbuf, acc.at[pl.ds(base + k * ZROWS, ZROWS)])
    plsc.subcore_barrier()

    def _slot(j, k, start_g, wait_s):
        _drain(x2_hbm, rows[k], gs[k])
        pltpu.async_copy(rows[k], acc.at[idx_d.at[pl.ds(j * CHUNK, CHUNK)]],
                         ss[k], add=True)
        if start_g:
            bn = (k + NB - 1) % NB
            if wait_s:
                _drain(x2_hbm, rows[bn], ss[bn])
            _xform(j + NB - 1)
            pltpu.async_copy(
                x2_hbm.at[idx_s.at[pl.ds((j + NB - 1) * CHUNK, CHUNK)]],
                rows[bn], gs[bn])

    for k in range(NB):
        _slot(k, k, start_g=True, wait_s=(k >= 1))

    def _turn(t, _):
        for k in range(NB):
            _slot(t * NB + k, k, start_g=True, wait_s=True)
        return 0
    lax.fori_loop(1, NT - 1, _turn, 0)

    for k in range(NB):
        _slot((NT - 1) * NB + k, k, start_g=(k == 0), wait_s=True)
    for k in range(NB):
        _drain(x2_hbm, rows[k], ss[k])

    plsc.subcore_barrier()
    col = c * D_HALF

    @pl.when(s < NUM_TILES - 1)
    def _():
        pltpu.sync_copy(acc.at[pl.ds(base, ROWS_PER_TILE)],
                        out_hbm.at[pl.ds(base, ROWS_PER_TILE),
                                   pl.ds(col, D_HALF)])

    @pl.when(s == NUM_TILES - 1)
    def _():
        pltpu.sync_copy(acc.at[pl.ds(base, LAST_ROWS)],
                        out_hbm.at[pl.ds(base, LAST_ROWS),
                                   pl.ds(col, D_HALF)])


def kernel(edge_index, x):
    e1d = edge_index.reshape(2 * N_EDGES)
    x2 = x.reshape(2 * N_NODES, D_HALF)
    return _mp_kernel(e1d, x2)
